# Initial kernel scaffold; baseline (speedup 1.0000x reference)
import jax
import jax.numpy as jnp
from jax.experimental import pallas as pl


def kernel(x, Wih_f, Whh_f, bih_f, bhh_f, Wih_b, Whh_b, bih_b, bhh_b, gate_W, gate_b, W1, b1, W2, b2, expert_bias):
    raise NotImplementedError("write your pallas kernel here")


# speedup vs baseline: 4.9698x; 4.9698x over previous
import functools

import jax
import jax.numpy as jnp
from jax.experimental import pallas as pl
from jax.experimental.pallas import tpu as pltpu

B, T, D = 1, 2048, 768
H = 1536
H3 = 3 * H
G = 2 * H
E, FFN = 8, 1024
BAL_W, Z_W = 0.01, 0.001


def _scan_body(x_ref, wihT_ref, bih_ref, whhT_ref, bhh_ref, h0_ref, o_ref,
               h_ref, gis_ref, ghs_ref, *, C, reverse):
    i = pl.program_id(0)

    @pl.when(i == 0)
    def _():
        h_ref[...] = h0_ref[...]

    bih = bih_ref[...]
    bhh = bhh_ref[...]

    def step(t, carry):
        tt = (C - 1 - t) if reverse else t
        h = h_ref[...]
        xt = x_ref[pl.ds(tt, 1), :]
        gis_ref[...] = jnp.concatenate(
            [jnp.dot(xt, wihT_ref[:, j * H:(j + 1) * H],
                     preferred_element_type=jnp.float32) for j in range(3)],
            axis=1)
        ghs_ref[...] = jnp.concatenate(
            [jnp.dot(h, whhT_ref[:, j * H:(j + 1) * H],
                     preferred_element_type=jnp.float32) for j in range(3)],
            axis=1)
        gi = gis_ref[...] + bih
        gh = ghs_ref[...] + bhh
        r = jax.nn.sigmoid(gi[:, :H] + gh[:, :H])
        z = jax.nn.sigmoid(gi[:, H:2 * H] + gh[:, H:2 * H])
        n = jnp.tanh(gi[:, 2 * H:] + r * gh[:, 2 * H:])
        h_new = (1.0 - z) * n + z * h
        h_ref[...] = h_new
        o_ref[pl.ds(tt, 1), :] = jnp.where(h_new >= 0, h_new, 0.01 * h_new)
        return carry

    jax.lax.fori_loop(0, C, step, 0)


def _scan_call(xf, wihT, bih, whhT, bhh, h0, *, reverse):
    C = 256
    NC = T // C
    if reverse:
        tmap = lambda i: (NC - 1 - i, 0)
    else:
        tmap = lambda i: (i, 0)
    return pl.pallas_call(
        functools.partial(_scan_body, C=C, reverse=reverse),
        grid=(NC,),
        in_specs=[
            pl.BlockSpec((C, D), tmap),
            pl.BlockSpec((D, H3), lambda i: (0, 0),
                         pipeline_mode=pl.Buffered(buffer_count=1)),
            pl.BlockSpec((1, H3), lambda i: (0, 0)),
            pl.BlockSpec((H, H3), lambda i: (0, 0),
                         pipeline_mode=pl.Buffered(buffer_count=1)),
            pl.BlockSpec((1, H3), lambda i: (0, 0)),
            pl.BlockSpec((1, H), lambda i: (0, 0)),
        ],
        out_specs=pl.BlockSpec((C, H), tmap),
        out_shape=jax.ShapeDtypeStruct((T, H), jnp.float32),
        scratch_shapes=[
            pltpu.VMEM((1, H), jnp.float32),
            pltpu.VMEM((1, H3), jnp.float32),
            pltpu.VMEM((1, H3), jnp.float32),
        ],
    )(xf, wihT, bih, whhT, bhh, h0)


def _router_body(flat_ref, gwT_ref, gb_ref, eb_ref, comb_ref, aux_ref):
    logits = (
        jnp.dot(flat_ref[...], gwT_ref[...], preferred_element_type=jnp.float32)
        + gb_ref[...]
        + eb_ref[...]
    )
    m = jnp.max(logits, axis=-1, keepdims=True)
    ex = jnp.exp(logits - m)
    s = jnp.sum(ex, axis=-1, keepdims=True)
    scores = ex / s
    lse = m + jnp.log(s)
    pos = jax.lax.broadcasted_iota(jnp.int32, scores.shape, 1)
    m1 = jnp.max(scores, axis=-1, keepdims=True)
    p1 = jnp.min(jnp.where(scores == m1, pos, E), axis=-1, keepdims=True)
    sel1 = pos == p1
    s2 = jnp.where(sel1, -1.0, scores)
    m2 = jnp.max(s2, axis=-1, keepdims=True)
    p2 = jnp.min(jnp.where(s2 == m2, pos, E), axis=-1, keepdims=True)
    sel2 = pos == p2
    denom = m1 + m2
    comb = jnp.where(sel1, m1 / denom, 0.0) + jnp.where(sel2, m2 / denom, 0.0)
    comb_ref[...] = comb
    mask = sel1.astype(jnp.float32) + sel2.astype(jnp.float32)
    fsum = jnp.sum(mask, axis=0)
    psum = jnp.sum(scores, axis=0)
    zsum = jnp.sum(lse * lse)
    n = jnp.float32(T)
    aux_ref[0, 0] = BAL_W * E * jnp.sum(fsum * psum) / (n * n) + Z_W * zsum / n


def _router_call(flat, gwT, gb, eb):
    return pl.pallas_call(
        _router_body,
        in_specs=[
            pl.BlockSpec((T, G), lambda: (0, 0)),
            pl.BlockSpec((G, E), lambda: (0, 0)),
            pl.BlockSpec((1, E), lambda: (0, 0)),
            pl.BlockSpec((1, E), lambda: (0, 0)),
        ],
        out_specs=[
            pl.BlockSpec((T, E), lambda: (0, 0)),
            pl.BlockSpec(memory_space=pltpu.SMEM),
        ],
        out_shape=[
            jax.ShapeDtypeStruct((T, E), jnp.float32),
            jax.ShapeDtypeStruct((1, 1), jnp.float32),
        ],
    )(flat, gwT, gb, eb)


def _moe_body(flat_ref, comb_ref, w1T_ref, b1_ref, w2T_ref, b2_ref, o_ref):
    e = pl.program_id(1)
    h = (
        jnp.dot(flat_ref[...], w1T_ref[0], preferred_element_type=jnp.float32)
        + b1_ref[0]
    )
    h = 0.5 * h * (1.0 + jax.lax.erf(h * 0.7071067811865476))
    eo = (
        jnp.dot(h, w2T_ref[0], preferred_element_type=jnp.float32)
        + b2_ref[0]
    )
    onehot = (jax.lax.broadcasted_iota(jnp.int32, (1, E), 1) == e).astype(
        jnp.float32
    )
    c = jnp.sum(comb_ref[...] * onehot, axis=-1, keepdims=True)
    contrib = c * eo

    @pl.when(e == 0)
    def _():
        o_ref[...] = contrib

    @pl.when(e > 0)
    def _():
        o_ref[...] += contrib


def _moe_call(flat, comb, w1T, b1r, w2T, b2r):
    NT = 4
    TB = T // NT
    return pl.pallas_call(
        _moe_body,
        grid=(NT, E),
        in_specs=[
            pl.BlockSpec((TB, G), lambda i, e: (i, 0)),
            pl.BlockSpec((TB, E), lambda i, e: (i, 0)),
            pl.BlockSpec((1, G, FFN), lambda i, e: (e, 0, 0)),
            pl.BlockSpec((1, 1, FFN), lambda i, e: (e, 0, 0)),
            pl.BlockSpec((1, FFN, D), lambda i, e: (e, 0, 0)),
            pl.BlockSpec((1, 1, D), lambda i, e: (e, 0, 0)),
        ],
        out_specs=pl.BlockSpec((TB, D), lambda i, e: (i, 0)),
        out_shape=jax.ShapeDtypeStruct((T, D), jnp.float32),
    )(flat, comb, w1T, b1r, w2T, b2r)


def kernel(x, Wih_f, Whh_f, bih_f, bhh_f, Wih_b, Whh_b, bih_b, bhh_b,
           gate_W, gate_b, W1, b1, W2, b2, expert_bias):
    xf = x.reshape(T, D)
    h0 = jnp.zeros((1, H), jnp.float32)
    out_f = _scan_call(xf, Wih_f.T, bih_f.reshape(1, H3),
                       Whh_f.T, bhh_f.reshape(1, H3), h0, reverse=False)
    out_b = _scan_call(xf, Wih_b.T, bih_b.reshape(1, H3),
                       Whh_b.T, bhh_b.reshape(1, H3), h0, reverse=True)
    flat = jnp.concatenate([out_f, out_b], axis=1)

    comb, aux = _router_call(flat, gate_W.T, gate_b.reshape(1, E),
                             expert_bias.reshape(1, E))

    w1T = jnp.transpose(W1, (0, 2, 1))
    w2T = jnp.transpose(W2, (0, 2, 1))
    out = _moe_call(flat, comb, w1T, b1.reshape(E, 1, FFN),
                    w2T, b2.reshape(E, 1, D))
    return out.reshape(B, T, D), aux.reshape(())
